# baseline (speedup 1.0000x reference)
import jax
import jax.numpy as jnp
from jax.experimental import pallas as pl
from jax.experimental.pallas import tpu as pltpu

_L_COORD = 5.0
_L_OBJ = 1.0
_L_NOOBJ = 0.5


def _corners(x, y, w_off, h_off):
    w = w_off * w_off
    h = h_off * h_off
    x1 = x - w / 2.0
    y1 = y - h / 2.0
    x2 = x1 + w
    y2 = y1 + h
    return x1, y1, x2, y2


def _iou(t, p):
    tx1, ty1, tx2, ty2 = t
    px1, py1, px2, py2 = p
    ltx = jnp.maximum(tx1, px1)
    lty = jnp.maximum(ty1, py1)
    rbx = jnp.minimum(tx2, px2)
    rby = jnp.minimum(ty2, py2)
    iw = jnp.maximum(rbx - ltx, 0.0)
    ih = jnp.maximum(rby - lty, 0.0)
    inter = iw * ih
    area_t = (tx2 - tx1) * (ty2 - ty1)
    area_p = (px2 - px1) * (py2 - py1)
    return inter / (area_t + area_p - inter)


def _loss_body(x, m, neg_acc, resp_acc, off_acc, cls_acc):
    d = x - m
    sq = d * d

    cls_cell = jnp.sum(sq[10:30], axis=0)

    m4 = m[4]
    m9 = m[9]
    x4 = x[4]
    x9 = x[9]

    neg_cell = (jnp.where(m4 < 1.0, sq[4], 0.0)
                + jnp.where(m9 < 1.0, sq[9], 0.0))

    iou1 = _iou(_corners(m[0], m[1], m[2], m[3]),
                _corners(x[0], x[1], x[2], x[3]))
    iou2 = _iou(_corners(m[5], m[6], m[7], m[8]),
                _corners(x[5], x[6], x[7], x[8]))

    sel2 = iou2 > iou1
    resp_sel = jnp.where(sel2, x9, x4)
    iou_sel = jnp.where(sel2, iou2, iou1)
    resp_cell = (resp_sel - iou_sel) ** 2

    off1 = sq[0] + sq[1] + sq[2] + sq[3]
    off2 = sq[5] + sq[6] + sq[7] + sq[8]
    off_cell = jnp.where(sel2, off2, off1)

    keep = (m4 + m9) > 0.9
    zero = jnp.zeros_like(cls_cell)

    neg_acc[...] += neg_cell
    resp_acc[...] += jnp.where(keep, resp_cell, zero)
    off_acc[...] += jnp.where(keep, off_cell, zero)
    cls_acc[...] += jnp.where(keep, cls_cell, zero)


def _stage1_kernel(pred_ref, meta_ref, out_ref,
                   neg_acc, resp_acc, off_acc, cls_acc):
    i = pl.program_id(0)
    n = pl.num_programs(0)

    @pl.when(i == 0)
    def _init():
        zero = jnp.zeros_like(neg_acc)
        neg_acc[...] = zero
        resp_acc[...] = zero
        off_acc[...] = zero
        cls_acc[...] = zero

    _loss_body(pred_ref[...], meta_ref[...],
               neg_acc, resp_acc, off_acc, cls_acc)

    @pl.when(i == n - 1)
    def _finalize():
        out_ref[0] = jnp.sum(neg_acc[...])
        out_ref[1] = jnp.sum(resp_acc[...])
        out_ref[2] = jnp.sum(off_acc[...])
        out_ref[3] = jnp.sum(cls_acc[...])


def _stage2_kernel(pred_ref, meta_ref, part_ref, out_ref,
                   neg_acc, resp_acc, off_acc, cls_acc):
    i = pl.program_id(0)
    n = pl.num_programs(0)

    @pl.when(i == 0)
    def _init():
        zero = jnp.zeros_like(neg_acc)
        neg_acc[...] = zero
        resp_acc[...] = zero
        off_acc[...] = zero
        cls_acc[...] = zero

    _loss_body(pred_ref[...], meta_ref[...],
               neg_acc, resp_acc, off_acc, cls_acc)

    @pl.when(i == n - 1)
    def _finalize():
        b_size = 128.0
        loss_neg = (part_ref[0] + jnp.sum(neg_acc[...])) / b_size * _L_NOOBJ
        loss_resp = (part_ref[1] + jnp.sum(resp_acc[...])) / b_size * _L_OBJ
        loss_off = (part_ref[2] + jnp.sum(off_acc[...])) / b_size * _L_COORD
        loss_cls = (part_ref[3] + jnp.sum(cls_acc[...])) / b_size
        out_ref[0] = loss_neg + loss_resp + loss_off + loss_cls
        out_ref[1] = loss_resp
        out_ref[2] = loss_neg
        out_ref[3] = loss_cls
        out_ref[4] = loss_off


def _relayout(a, groups, lanes, c):
    return jnp.transpose(a.reshape(groups, lanes, c), (2, 0, 1))


def kernel(pred, meta):
    b, h, w, c = pred.shape
    lanes = 128
    half_groups = (b // 2) * h * w // lanes
    g_blk = 224
    grid = half_groups // g_blk

    halves = []
    for lo, hi in ((0, b // 2), (b // 2, b)):
        pc = _relayout(pred[lo:hi], half_groups, lanes, c)
        mc = _relayout(meta[lo:hi], half_groups, lanes, c)
        halves.append((pc, mc))

    specs = [
        pl.BlockSpec((c, g_blk, lanes), lambda i: (0, i, 0)),
        pl.BlockSpec((c, g_blk, lanes), lambda i: (0, i, 0)),
    ]

    def scratch():
        return [pltpu.VMEM((g_blk, lanes), jnp.float32) for _ in range(4)]

    part = pl.pallas_call(
        _stage1_kernel,
        grid=(grid,),
        in_specs=specs,
        out_specs=pl.BlockSpec(memory_space=pltpu.SMEM),
        out_shape=jax.ShapeDtypeStruct((4,), jnp.float32),
        scratch_shapes=scratch(),
    )(*halves[0])

    out = pl.pallas_call(
        _stage2_kernel,
        grid=(grid,),
        in_specs=specs + [pl.BlockSpec(memory_space=pltpu.SMEM)],
        out_specs=pl.BlockSpec(memory_space=pltpu.SMEM),
        out_shape=jax.ShapeDtypeStruct((5,), jnp.float32),
        scratch_shapes=scratch(),
    )(*halves[1], part)

    return (out[0].reshape(()), out[1].reshape(()), out[2].reshape(()),
            out[3].reshape(()), out[4].reshape(()))


# speedup vs baseline: 1.7350x; 1.7350x over previous
import jax
import jax.numpy as jnp
from jax.experimental import pallas as pl
from jax.experimental.pallas import tpu as pltpu

_L_COORD = 5.0
_L_OBJ = 1.0
_L_NOOBJ = 0.5


def _corners(x, y, w_off, h_off):
    w = w_off * w_off
    h = h_off * h_off
    x1 = x - w / 2.0
    y1 = y - h / 2.0
    x2 = x1 + w
    y2 = y1 + h
    return x1, y1, x2, y2


def _iou(t, p):
    tx1, ty1, tx2, ty2 = t
    px1, py1, px2, py2 = p
    ltx = jnp.maximum(tx1, px1)
    lty = jnp.maximum(ty1, py1)
    rbx = jnp.minimum(tx2, px2)
    rby = jnp.minimum(ty2, py2)
    iw = jnp.maximum(rbx - ltx, 0.0)
    ih = jnp.maximum(rby - lty, 0.0)
    inter = iw * ih
    area_t = (tx2 - tx1) * (ty2 - ty1)
    area_p = (px2 - px1) * (py2 - py1)
    return inter / (area_t + area_p - inter)


def _loss_kernel(pred_ref, meta_ref, out_ref,
                 neg_acc, resp_acc, off_acc, cls_acc):
    i = pl.program_id(0)
    n = pl.num_programs(0)

    @pl.when(i == 0)
    def _init():
        zero = jnp.zeros_like(neg_acc)
        neg_acc[...] = zero
        resp_acc[...] = zero
        off_acc[...] = zero
        cls_acc[...] = zero

    x = pred_ref[...]
    m = meta_ref[...]

    d = x - m
    sq = d * d

    cls_cell = jnp.sum(sq[10:30], axis=0)

    m4 = m[4]
    m9 = m[9]
    x4 = x[4]
    x9 = x[9]

    neg_cell = (jnp.where(m4 < 1.0, sq[4], 0.0)
                + jnp.where(m9 < 1.0, sq[9], 0.0))

    t1 = _corners(m[0], m[1], m[2], m[3])
    p1 = _corners(x[0], x[1], x[2], x[3])
    t2 = _corners(m[5], m[6], m[7], m[8])
    p2 = _corners(x[5], x[6], x[7], x[8])
    iou1 = _iou(t1, p1)
    iou2 = _iou(t2, p2)

    sel2 = iou2 > iou1
    resp_sel = jnp.where(sel2, x9, x4)
    iou_sel = jnp.where(sel2, iou2, iou1)
    resp_cell = (resp_sel - iou_sel) ** 2

    off1 = sq[0] + sq[1] + sq[2] + sq[3]
    off2 = sq[5] + sq[6] + sq[7] + sq[8]
    off_cell = jnp.where(sel2, off2, off1)

    keep = (m4 + m9) > 0.9
    zero = jnp.zeros_like(cls_cell)

    neg_acc[...] += neg_cell
    resp_acc[...] += jnp.where(keep, resp_cell, zero)
    off_acc[...] += jnp.where(keep, off_cell, zero)
    cls_acc[...] += jnp.where(keep, cls_cell, zero)

    @pl.when(i == n - 1)
    def _finalize():
        b_size = 128.0
        loss_neg = jnp.sum(neg_acc[...]) / b_size * _L_NOOBJ
        loss_resp = jnp.sum(resp_acc[...]) / b_size * _L_OBJ
        loss_off = jnp.sum(off_acc[...]) / b_size * _L_COORD
        loss_cls = jnp.sum(cls_acc[...]) / b_size
        out_ref[0] = loss_neg + loss_resp + loss_off + loss_cls
        out_ref[1] = loss_resp
        out_ref[2] = loss_neg
        out_ref[3] = loss_cls
        out_ref[4] = loss_off


def kernel(pred, meta):
    b, h, w, c = pred.shape
    cells = b * h * w
    lanes = 128
    groups = cells // lanes
    g_blk = 224
    grid = groups // g_blk

    pc = jnp.transpose(pred.reshape(groups, lanes, c), (2, 0, 1))
    mc = jnp.transpose(meta.reshape(groups, lanes, c), (2, 0, 1))

    out = pl.pallas_call(
        _loss_kernel,
        grid=(grid,),
        in_specs=[
            pl.BlockSpec((c, g_blk, lanes), lambda i: (0, i, 0)),
            pl.BlockSpec((c, g_blk, lanes), lambda i: (0, i, 0)),
        ],
        out_specs=pl.BlockSpec(memory_space=pltpu.SMEM),
        out_shape=jax.ShapeDtypeStruct((5,), jnp.float32),
        scratch_shapes=[pltpu.VMEM((g_blk, lanes), jnp.float32)
                        for _ in range(4)],
    )(pc, mc)
    return (out[0].reshape(()), out[1].reshape(()), out[2].reshape(()),
            out[3].reshape(()), out[4].reshape(()))
